# Initial kernel scaffold; baseline (speedup 1.0000x reference)
import jax
import jax.numpy as jnp
from jax.experimental import pallas as pl


def kernel(z, pos, edge_index, batch, low_params, dif_params, corr_w):
    raise NotImplementedError("write your pallas kernel here")


# speedup vs baseline: 1.4727x; 1.4727x over previous
import functools
import math

import jax
import jax.numpy as jnp
from jax import lax
from jax.experimental import pallas as pl
from jax.experimental.pallas import tpu as pltpu
from jax.experimental.pallas import tpu_sc as plsc

N_NODES = 10000
N_EDGES = 320000
N_MOL = 512
CUTOFF = 6.0
NG = 50
NGP = 64
NC = 2
NS = 16
NW = NC * NS

_f32 = jnp.float32


def _ssp(x):
    return jnp.maximum(x, 0.0) + jnp.log(1.0 + jnp.exp(-jnp.abs(x))) - math.log(2.0)


def _mesh():
    return plsc.VectorSubcoreMesh(core_axis_name="c", subcore_axis_name="s")


@functools.lru_cache(maxsize=None)
def _sc_gather(n_rows, d, n_idx, k):
    ew = n_idx // NW
    steps = ew // k
    assert ew % k == 0 and k % 8 == 0 and ew * NW == n_idx

    def body(table_hbm, idx_hbm, out_hbm, idx_v, rows_v, sem):
        wid = lax.axis_index("s") * NC + lax.axis_index("c")
        base = wid * ew

        def step(i, c):
            off = base + i * k
            pltpu.sync_copy(idx_hbm.at[pl.ds(off, k)], idx_v)
            pltpu.async_copy(table_hbm.at[idx_v], rows_v, sem).wait()
            pltpu.sync_copy(rows_v, out_hbm.at[pl.ds(off, k)])
            return c

        lax.fori_loop(0, steps, step, 0)

    return pl.kernel(
        body,
        out_type=jax.ShapeDtypeStruct((n_idx, d), _f32),
        mesh=_mesh(),
        compiler_params=pltpu.CompilerParams(use_tc_tiling_on_sc=False),
        scratch_types=[
            pltpu.VMEM((k,), jnp.int32),
            pltpu.VMEM((k, d), _f32),
            pltpu.SemaphoreType.DMA,
        ],
    )


@functools.lru_cache(maxsize=None)
def _sc_scatter_add(n_rows, d, n_idx, k):
    ew = n_idx // NW
    steps = ew // k
    zr = n_rows // NS
    zc = 25
    zsteps = zr // zc
    assert ew % k == 0 and k % 8 == 0 and zr * NS == n_rows and zr % zc == 0

    def body(msg_hbm, dst_hbm, out_hbm, idx_v, rows_v, zbuf, accum, sem):
        cid = lax.axis_index("c")
        sid = lax.axis_index("s")
        wid = sid * NC + cid

        def zrow(r, c):
            for j in range(d // 16):
                zbuf[r, pl.ds(j * 16, 16)] = jnp.zeros((16,), _f32)
            return c

        lax.fori_loop(0, zc, zrow, 0)
        for j in range(zsteps):
            pltpu.sync_copy(zbuf, accum.at[pl.ds(sid * zr + j * zc, zc)])
        plsc.subcore_barrier()

        def step(i, c):
            off = wid * ew + i * k
            pltpu.sync_copy(dst_hbm.at[pl.ds(off, k)], idx_v)
            pltpu.sync_copy(msg_hbm.at[pl.ds(off, k)], rows_v)
            pltpu.sync_copy(rows_v, accum.at[idx_v], add=True)
            return c

        lax.fori_loop(0, steps, step, 0)
        plsc.subcore_barrier()

        pltpu.sync_copy(accum.at[pl.ds(sid * zr, zr)],
                        out_hbm.at[cid].at[pl.ds(sid * zr, zr)])

    return pl.kernel(
        body,
        out_type=jax.ShapeDtypeStruct((NC, n_rows, d), _f32),
        mesh=_mesh(),
        compiler_params=pltpu.CompilerParams(use_tc_tiling_on_sc=False),
        scratch_types=[
            pltpu.VMEM((k,), jnp.int32),
            pltpu.VMEM((k, d), _f32),
            pltpu.VMEM((zc, d), _f32),
            pltpu.VMEM_SHARED((n_rows, d), _f32),
            pltpu.SemaphoreType.DMA,
        ],
    )


@functools.lru_cache(maxsize=None)
def _embed(h):
    r = 1000
    grid = N_NODES // r
    zp = 104

    def body(z_r, emb_r, l1_r, h_r, hlin_r):
        oh = (z_r[...] == lax.broadcasted_iota(jnp.int32, (r, zp), 1)
              ).astype(_f32)
        h0 = jnp.dot(oh, emb_r[...], preferred_element_type=_f32)
        h_r[...] = h0
        hlin_r[...] = jnp.dot(h0, l1_r[...], preferred_element_type=_f32)

    return pl.pallas_call(
        body,
        grid=(grid,),
        in_specs=[
            pl.BlockSpec((r, 1), lambda i: (i, 0)),
            pl.BlockSpec((zp, h), lambda i: (0, 0)),
            pl.BlockSpec((h, h), lambda i: (0, 0)),
        ],
        out_specs=(pl.BlockSpec((r, h), lambda i: (i, 0)),
                   pl.BlockSpec((r, h), lambda i: (i, 0))),
        out_shape=(jax.ShapeDtypeStruct((N_NODES, h), _f32),
                   jax.ShapeDtypeStruct((N_NODES, h), _f32)),
    )


@functools.lru_cache(maxsize=None)
def _edge_msg(h):
    te = 2000
    grid = N_EDGES // te
    delta = CUTOFF / (NG - 1)
    coeff = -0.5 / delta ** 2

    def body(px_r, py_r, xj_r, w1_r, b1_r, w2_r, b2_r, out_r):
        dxyz = px_r[...] - py_r[...]
        d2 = jnp.sum(dxyz * dxyz, axis=1, keepdims=True)
        d = jnp.sqrt(d2 + 1e-12)
        offs = lax.broadcasted_iota(jnp.int32, (te, NGP), 1).astype(_f32) * delta
        dd = d - offs
        rbf = jnp.exp(coeff * (dd * dd))
        t = _ssp(jnp.dot(rbf, w1_r[...], preferred_element_type=_f32) + b1_r[...])
        w = jnp.dot(t, w2_r[...], preferred_element_type=_f32) + b2_r[...]
        c = 0.5 * (jnp.cos(d * (math.pi / CUTOFF)) + 1.0)
        c = jnp.where(d < CUTOFF, c, 0.0)
        out_r[...] = xj_r[...] * (w * c)

    return pl.pallas_call(
        body,
        grid=(grid,),
        in_specs=[
            pl.BlockSpec((te, 16), lambda i: (i, 0)),
            pl.BlockSpec((te, 16), lambda i: (i, 0)),
            pl.BlockSpec((te, h), lambda i: (i, 0)),
            pl.BlockSpec((NGP, h), lambda i: (0, 0)),
            pl.BlockSpec((1, h), lambda i: (0, 0)),
            pl.BlockSpec((h, h), lambda i: (0, 0)),
            pl.BlockSpec((1, h), lambda i: (0, 0)),
        ],
        out_specs=pl.BlockSpec((te, h), lambda i: (i, 0)),
        out_shape=jax.ShapeDtypeStruct((N_EDGES, h), _f32),
    )


@functools.lru_cache(maxsize=None)
def _node_update(h, nxt):
    r = 1000
    grid = N_NODES // r

    def body(a0_r, a1_r, h_r, l2w_r, l2b_r, lw_r, lb_r, *rest):
        agg = a0_r[...] + a1_r[...]
        x = _ssp(jnp.dot(agg, l2w_r[...], preferred_element_type=_f32)
                 + l2b_r[...])
        x = jnp.dot(x, lw_r[...], preferred_element_type=_f32) + lb_r[...]
        hn = h_r[...] + x
        if nxt:
            nw_r, hn_r, hlin_r = rest
            hn_r[...] = hn
            hlin_r[...] = jnp.dot(hn, nw_r[...], preferred_element_type=_f32)
        else:
            (hn_r,) = rest
            hn_r[...] = hn

    in_specs = [
        pl.BlockSpec((r, h), lambda i: (i, 0)),
        pl.BlockSpec((r, h), lambda i: (i, 0)),
        pl.BlockSpec((r, h), lambda i: (i, 0)),
        pl.BlockSpec((h, h), lambda i: (0, 0)),
        pl.BlockSpec((1, h), lambda i: (0, 0)),
        pl.BlockSpec((h, h), lambda i: (0, 0)),
        pl.BlockSpec((1, h), lambda i: (0, 0)),
    ]
    if nxt:
        in_specs.append(pl.BlockSpec((h, h), lambda i: (0, 0)))
        out_specs = (pl.BlockSpec((r, h), lambda i: (i, 0)),
                     pl.BlockSpec((r, h), lambda i: (i, 0)))
        out_shape = (jax.ShapeDtypeStruct((N_NODES, h), _f32),
                     jax.ShapeDtypeStruct((N_NODES, h), _f32))
    else:
        out_specs = pl.BlockSpec((r, h), lambda i: (i, 0))
        out_shape = jax.ShapeDtypeStruct((N_NODES, h), _f32)

    return pl.pallas_call(
        body,
        grid=(grid,),
        in_specs=in_specs,
        out_specs=out_specs,
        out_shape=out_shape,
    )


@functools.lru_cache(maxsize=None)
def _readout(h, with_prev):
    r = 1000
    grid = N_NODES // r
    hh = h // 2

    def body(h_r, w1_r, b1_r, w2_r, b2_r, bt_r, aux_r, out_r):
        i = pl.program_id(0)
        t = _ssp(jnp.dot(h_r[...], w1_r[...], preferred_element_type=_f32)
                 + b1_r[...])
        e = jnp.dot(t, w2_r[...], preferred_element_type=_f32) + b2_r[...]
        oh = (bt_r[...] == lax.broadcasted_iota(jnp.int32, (r, N_MOL), 1)
              ).astype(_f32)
        part = jnp.sum(oh * e, axis=0, keepdims=True)
        if with_prev:
            @pl.when(i == 0)
            def _():
                out_r[...] = aux_r[...]
            out_r[...] += part
        else:
            @pl.when(i == 0)
            def _():
                out_r[...] = jnp.zeros((1, N_MOL), _f32)
            out_r[...] += part * aux_r[0, 0]

    aux_spec = (pl.BlockSpec((1, N_MOL), lambda i: (0, 0)) if with_prev
                else pl.BlockSpec((1, 1), lambda i: (0, 0)))
    return pl.pallas_call(
        body,
        grid=(grid,),
        in_specs=[
            pl.BlockSpec((r, h), lambda i: (i, 0)),
            pl.BlockSpec((h, hh), lambda i: (0, 0)),
            pl.BlockSpec((1, hh), lambda i: (0, 0)),
            pl.BlockSpec((hh, 1), lambda i: (0, 0)),
            pl.BlockSpec((1, 1), lambda i: (0, 0)),
            pl.BlockSpec((r, 1), lambda i: (i, 0)),
            aux_spec,
        ],
        out_specs=pl.BlockSpec((1, N_MOL), lambda i: (0, 0)),
        out_shape=jax.ShapeDtypeStruct((1, N_MOL), _f32),
    )


def _run_model(params, hdim, z2, px, py, src, dst):
    blocks = params["blocks"]
    t_total = len(blocks)
    emb = jnp.pad(params["emb"], ((0, 4), (0, 0)))
    h, hlin = _embed(hdim)(z2, emb, blocks[0]["lin1_w"])
    for t, blk in enumerate(blocks):
        w1p = jnp.pad(blk["mlp_w1"], ((0, NGP - NG), (0, 0)))
        xj = _sc_gather(N_NODES, hdim, N_EDGES, 400)(hlin, src)
        msg = _edge_msg(hdim)(px, py, xj, w1p,
                              blk["mlp_b1"].reshape(1, -1), blk["mlp_w2"],
                              blk["mlp_b2"].reshape(1, -1))
        agg = _sc_scatter_add(N_NODES, hdim, N_EDGES, 200)(msg, dst)
        a0, a1 = agg[0], agg[1]
        args = (a0, a1, h, blk["lin2_w"], blk["lin2_b"].reshape(1, -1),
                blk["lin_w"], blk["lin_b"].reshape(1, -1))
        if t + 1 < t_total:
            h, hlin = _node_update(hdim, True)(*args,
                                               blocks[t + 1]["lin1_w"])
        else:
            h = _node_update(hdim, False)(*args)
    return h


def kernel(z, pos, edge_index, batch, low_params, dif_params, corr_w):
    src = edge_index[0].astype(jnp.int32)
    dst = edge_index[1].astype(jnp.int32)
    idx2 = jnp.concatenate([src, dst])
    pos16 = jnp.pad(pos.astype(_f32), ((0, 0), (0, 13)))
    pxy = _sc_gather(N_NODES, 16, 2 * N_EDGES, 2000)(pos16, idx2)
    px, py = pxy[:N_EDGES], pxy[N_EDGES:]
    z2 = z.reshape(-1, 1).astype(jnp.int32)
    b2 = batch.reshape(-1, 1).astype(jnp.int32)

    h_low = _run_model(low_params, 128, z2, px, py, src, dst)
    h_dif = _run_model(dif_params, 64, z2, px, py, src, dst)

    y0 = _readout(128, False)(h_low, low_params["out1_w"],
                              low_params["out1_b"].reshape(1, -1),
                              low_params["out2_w"],
                              low_params["out2_b"].reshape(1, -1),
                              b2, corr_w)
    y = _readout(64, True)(h_dif, dif_params["out1_w"],
                           dif_params["out1_b"].reshape(1, -1),
                           dif_params["out2_w"],
                           dif_params["out2_b"].reshape(1, -1),
                           b2, y0)
    return y.reshape(N_MOL)


# speedup vs baseline: 2.4483x; 1.6625x over previous
import functools
import math

import jax
import jax.numpy as jnp
from jax import lax
from jax.experimental import pallas as pl
from jax.experimental.pallas import tpu as pltpu
from jax.experimental.pallas import tpu_sc as plsc

N_NODES = 10000
N_EDGES = 320000
N_MOL = 512
CUTOFF = 6.0
NG = 50
NGP = 64
NC = 2
NS = 16
NW = NC * NS

_f32 = jnp.float32


def _ssp(x):
    return jnp.maximum(x, 0.0) + jnp.log(1.0 + jnp.exp(-jnp.abs(x))) - math.log(2.0)


def _mesh():
    return plsc.VectorSubcoreMesh(core_axis_name="c", subcore_axis_name="s")


@functools.lru_cache(maxsize=None)
def _sc_gather(n_rows, d, n_idx, k):
    ew = n_idx // NW
    steps = ew // k
    assert ew % k == 0 and k % 8 == 0 and ew * NW == n_idx

    def body(table_hbm, idx_hbm, out_hbm, idx_v, rows_v, sem):
        wid = lax.axis_index("s") * NC + lax.axis_index("c")
        base = wid * ew

        def step(i, c):
            off = base + i * k
            pltpu.sync_copy(idx_hbm.at[pl.ds(off, k)], idx_v)
            pltpu.async_copy(table_hbm.at[idx_v], rows_v, sem).wait()
            pltpu.sync_copy(rows_v, out_hbm.at[pl.ds(off, k)])
            return c

        lax.fori_loop(0, steps, step, 0)

    return pl.kernel(
        body,
        out_type=jax.ShapeDtypeStruct((n_idx, d), _f32),
        mesh=_mesh(),
        compiler_params=pltpu.CompilerParams(use_tc_tiling_on_sc=False),
        scratch_types=[
            pltpu.VMEM((k,), jnp.int32),
            pltpu.VMEM((k, d), _f32),
            pltpu.SemaphoreType.DMA,
        ],
    )


@functools.lru_cache(maxsize=None)
def _sc_msg_scatter(n_rows, d, n_idx, k):
    ew = n_idx // NW
    steps = ew // k
    zr = n_rows // NS
    zc = 25
    zsteps = zr // zc
    assert ew % k == 0 and k % 8 == 0 and zr * NS == n_rows and zr % zc == 0

    def body(w_hbm, hlin_hbm, src_hbm, dst_hbm, out_hbm,
             sidx_v, didx_v, xrows_v, wrows_v, zbuf, accum, sem):
        cid = lax.axis_index("c")
        sid = lax.axis_index("s")
        wid = sid * NC + cid

        def zrow(r, c):
            for j in range(d // 16):
                zbuf[r, pl.ds(j * 16, 16)] = jnp.zeros((16,), _f32)
            return c

        lax.fori_loop(0, zc, zrow, 0)
        for j in range(zsteps):
            pltpu.sync_copy(zbuf, accum.at[pl.ds(sid * zr + j * zc, zc)])
        plsc.subcore_barrier()

        def step(i, c):
            off = wid * ew + i * k
            pltpu.sync_copy(src_hbm.at[pl.ds(off, k)], sidx_v)
            pltpu.sync_copy(dst_hbm.at[pl.ds(off, k)], didx_v)
            gat = pltpu.async_copy(hlin_hbm.at[sidx_v], xrows_v, sem)
            pltpu.sync_copy(w_hbm.at[pl.ds(off, k)], wrows_v)
            gat.wait()

            def mrow(r, c2):
                for j in range(d // 16):
                    sl = pl.ds(j * 16, 16)
                    wrows_v[r, sl] = wrows_v[r, sl] * xrows_v[r, sl]
                return c2

            lax.fori_loop(0, k, mrow, 0)
            pltpu.sync_copy(wrows_v, accum.at[didx_v], add=True)
            return c

        lax.fori_loop(0, steps, step, 0)
        plsc.subcore_barrier()

        pltpu.sync_copy(accum.at[pl.ds(sid * zr, zr)],
                        out_hbm.at[cid].at[pl.ds(sid * zr, zr)])

    return pl.kernel(
        body,
        out_type=jax.ShapeDtypeStruct((NC, n_rows, d), _f32),
        mesh=_mesh(),
        compiler_params=pltpu.CompilerParams(use_tc_tiling_on_sc=False),
        scratch_types=[
            pltpu.VMEM((k,), jnp.int32),
            pltpu.VMEM((k,), jnp.int32),
            pltpu.VMEM((k, d), _f32),
            pltpu.VMEM((k, d), _f32),
            pltpu.VMEM((zc, d), _f32),
            pltpu.VMEM_SHARED((n_rows, d), _f32),
            pltpu.SemaphoreType.DMA,
        ],
    )


@functools.lru_cache(maxsize=None)
def _embed(h):
    r = 1000
    grid = N_NODES // r
    zp = 104

    def body(z_r, emb_r, l1_r, h_r, hlin_r):
        oh = (z_r[...] == lax.broadcasted_iota(jnp.int32, (r, zp), 1)
              ).astype(_f32)
        h0 = jnp.dot(oh, emb_r[...], preferred_element_type=_f32)
        h_r[...] = h0
        hlin_r[...] = jnp.dot(h0, l1_r[...], preferred_element_type=_f32)

    return pl.pallas_call(
        body,
        grid=(grid,),
        in_specs=[
            pl.BlockSpec((r, 1), lambda i: (i, 0)),
            pl.BlockSpec((zp, h), lambda i: (0, 0)),
            pl.BlockSpec((h, h), lambda i: (0, 0)),
        ],
        out_specs=(pl.BlockSpec((r, h), lambda i: (i, 0)),
                   pl.BlockSpec((r, h), lambda i: (i, 0))),
        out_shape=(jax.ShapeDtypeStruct((N_NODES, h), _f32),
                   jax.ShapeDtypeStruct((N_NODES, h), _f32)),
    )


@functools.lru_cache(maxsize=None)
def _edge_wall(hdims):
    te = 2000
    grid = N_EDGES // te
    delta = CUTOFF / (NG - 1)
    coeff = -0.5 / delta ** 2
    nb = len(hdims)

    def body(*refs):
        px_r, py_r = refs[0], refs[1]
        ins = refs[2:2 + 4 * nb]
        outs = refs[2 + 4 * nb:]
        dxyz = px_r[...] - py_r[...]
        d2 = jnp.sum(dxyz * dxyz, axis=1, keepdims=True)
        d = jnp.sqrt(d2 + 1e-12)
        offs = lax.broadcasted_iota(jnp.int32, (te, NGP), 1).astype(_f32) * delta
        dd = d - offs
        rbf = jnp.exp(coeff * (dd * dd))
        c = 0.5 * (jnp.cos(d * (math.pi / CUTOFF)) + 1.0)
        c = jnp.where(d < CUTOFF, c, 0.0)
        for bi in range(nb):
            w1_r, b1_r, w2_r, b2_r = ins[4 * bi:4 * bi + 4]
            t = _ssp(jnp.dot(rbf, w1_r[...], preferred_element_type=_f32)
                     + b1_r[...])
            w = jnp.dot(t, w2_r[...], preferred_element_type=_f32) + b2_r[...]
            outs[bi][...] = w * c

    in_specs = [pl.BlockSpec((te, 16), lambda i: (i, 0)),
                pl.BlockSpec((te, 16), lambda i: (i, 0))]
    for h in hdims:
        in_specs += [
            pl.BlockSpec((NGP, h), lambda i: (0, 0)),
            pl.BlockSpec((1, h), lambda i: (0, 0)),
            pl.BlockSpec((h, h), lambda i: (0, 0)),
            pl.BlockSpec((1, h), lambda i: (0, 0)),
        ]
    return pl.pallas_call(
        body,
        grid=(grid,),
        in_specs=in_specs,
        out_specs=tuple(pl.BlockSpec((te, h), lambda i: (i, 0))
                        for h in hdims),
        out_shape=tuple(jax.ShapeDtypeStruct((N_EDGES, h), _f32)
                        for h in hdims),
    )


@functools.lru_cache(maxsize=None)
def _node_update(h, nxt):
    r = 1000
    grid = N_NODES // r

    def body(a0_r, a1_r, h_r, l2w_r, l2b_r, lw_r, lb_r, *rest):
        agg = a0_r[...] + a1_r[...]
        x = _ssp(jnp.dot(agg, l2w_r[...], preferred_element_type=_f32)
                 + l2b_r[...])
        x = jnp.dot(x, lw_r[...], preferred_element_type=_f32) + lb_r[...]
        hn = h_r[...] + x
        if nxt:
            nw_r, hn_r, hlin_r = rest
            hn_r[...] = hn
            hlin_r[...] = jnp.dot(hn, nw_r[...], preferred_element_type=_f32)
        else:
            (hn_r,) = rest
            hn_r[...] = hn

    in_specs = [
        pl.BlockSpec((r, h), lambda i: (i, 0)),
        pl.BlockSpec((r, h), lambda i: (i, 0)),
        pl.BlockSpec((r, h), lambda i: (i, 0)),
        pl.BlockSpec((h, h), lambda i: (0, 0)),
        pl.BlockSpec((1, h), lambda i: (0, 0)),
        pl.BlockSpec((h, h), lambda i: (0, 0)),
        pl.BlockSpec((1, h), lambda i: (0, 0)),
    ]
    if nxt:
        in_specs.append(pl.BlockSpec((h, h), lambda i: (0, 0)))
        out_specs = (pl.BlockSpec((r, h), lambda i: (i, 0)),
                     pl.BlockSpec((r, h), lambda i: (i, 0)))
        out_shape = (jax.ShapeDtypeStruct((N_NODES, h), _f32),
                     jax.ShapeDtypeStruct((N_NODES, h), _f32))
    else:
        out_specs = pl.BlockSpec((r, h), lambda i: (i, 0))
        out_shape = jax.ShapeDtypeStruct((N_NODES, h), _f32)

    return pl.pallas_call(
        body,
        grid=(grid,),
        in_specs=in_specs,
        out_specs=out_specs,
        out_shape=out_shape,
    )


@functools.lru_cache(maxsize=None)
def _readout(h, with_prev):
    r = 1000
    grid = N_NODES // r
    hh = h // 2

    def body(h_r, w1_r, b1_r, w2_r, b2_r, bt_r, aux_r, out_r):
        i = pl.program_id(0)
        t = _ssp(jnp.dot(h_r[...], w1_r[...], preferred_element_type=_f32)
                 + b1_r[...])
        e = jnp.dot(t, w2_r[...], preferred_element_type=_f32) + b2_r[...]
        oh = (bt_r[...] == lax.broadcasted_iota(jnp.int32, (r, N_MOL), 1)
              ).astype(_f32)
        part = jnp.sum(oh * e, axis=0, keepdims=True)
        if with_prev:
            @pl.when(i == 0)
            def _():
                out_r[...] = aux_r[...]
            out_r[...] += part
        else:
            @pl.when(i == 0)
            def _():
                out_r[...] = jnp.zeros((1, N_MOL), _f32)
            out_r[...] += part * aux_r[0, 0]

    aux_spec = (pl.BlockSpec((1, N_MOL), lambda i: (0, 0)) if with_prev
                else pl.BlockSpec((1, 1), lambda i: (0, 0)))
    return pl.pallas_call(
        body,
        grid=(grid,),
        in_specs=[
            pl.BlockSpec((r, h), lambda i: (i, 0)),
            pl.BlockSpec((h, hh), lambda i: (0, 0)),
            pl.BlockSpec((1, hh), lambda i: (0, 0)),
            pl.BlockSpec((hh, 1), lambda i: (0, 0)),
            pl.BlockSpec((1, 1), lambda i: (0, 0)),
            pl.BlockSpec((r, 1), lambda i: (i, 0)),
            aux_spec,
        ],
        out_specs=pl.BlockSpec((1, N_MOL), lambda i: (0, 0)),
        out_shape=jax.ShapeDtypeStruct((1, N_MOL), _f32),
    )


def _run_model(params, hdim, z2, ws, src, dst, sc_k):
    blocks = params["blocks"]
    t_total = len(blocks)
    emb = jnp.pad(params["emb"], ((0, 4), (0, 0)))
    h, hlin = _embed(hdim)(z2, emb, blocks[0]["lin1_w"])
    for t, blk in enumerate(blocks):
        agg = _sc_msg_scatter(N_NODES, hdim, N_EDGES, sc_k)(
            ws[t], hlin, src, dst)
        a0, a1 = agg[0], agg[1]
        args = (a0, a1, h, blk["lin2_w"], blk["lin2_b"].reshape(1, -1),
                blk["lin_w"], blk["lin_b"].reshape(1, -1))
        if t + 1 < t_total:
            h, hlin = _node_update(hdim, True)(*args,
                                               blocks[t + 1]["lin1_w"])
        else:
            h = _node_update(hdim, False)(*args)
    return h


def kernel(z, pos, edge_index, batch, low_params, dif_params, corr_w):
    src = edge_index[0].astype(jnp.int32)
    dst = edge_index[1].astype(jnp.int32)
    idx2 = jnp.concatenate([src, dst])
    pos16 = jnp.pad(pos.astype(_f32), ((0, 0), (0, 13)))
    pxy = _sc_gather(N_NODES, 16, 2 * N_EDGES, 2000)(pos16, idx2)
    px, py = pxy[:N_EDGES], pxy[N_EDGES:]
    z2 = z.reshape(-1, 1).astype(jnp.int32)
    b2 = batch.reshape(-1, 1).astype(jnp.int32)

    all_blocks = low_params["blocks"] + dif_params["blocks"]
    hdims = tuple(blk["mlp_w2"].shape[0] for blk in all_blocks)
    wall_in = []
    for blk in all_blocks:
        wall_in += [jnp.pad(blk["mlp_w1"], ((0, NGP - NG), (0, 0))),
                    blk["mlp_b1"].reshape(1, -1), blk["mlp_w2"],
                    blk["mlp_b2"].reshape(1, -1)]
    ws = _edge_wall(hdims)(px, py, *wall_in)

    h_low = _run_model(low_params, 128, z2, ws[:3], src, dst, 80)
    h_dif = _run_model(dif_params, 64, z2, ws[3:], src, dst, 200)

    y0 = _readout(128, False)(h_low, low_params["out1_w"],
                              low_params["out1_b"].reshape(1, -1),
                              low_params["out2_w"],
                              low_params["out2_b"].reshape(1, -1),
                              b2, corr_w)
    y = _readout(64, True)(h_dif, dif_params["out1_w"],
                           dif_params["out1_b"].reshape(1, -1),
                           dif_params["out2_w"],
                           dif_params["out2_b"].reshape(1, -1),
                           b2, y0)
    return y.reshape(N_MOL)


# speedup vs baseline: 3.1671x; 1.2936x over previous
import functools
import math

import jax
import jax.numpy as jnp
from jax import lax
from jax.experimental import pallas as pl
from jax.experimental.pallas import tpu as pltpu
from jax.experimental.pallas import tpu_sc as plsc

N_NODES = 10000
N_EDGES = 320000
N_MOL = 512
CUTOFF = 6.0
NG = 50
NGP = 64
NC = 2
NS = 16
NW = NC * NS

_f32 = jnp.float32


def _ssp(x):
    return jnp.maximum(x, 0.0) + jnp.log(1.0 + jnp.exp(-jnp.abs(x))) - math.log(2.0)


def _mesh():
    return plsc.VectorSubcoreMesh(core_axis_name="c", subcore_axis_name="s")


@functools.lru_cache(maxsize=None)
def _sc_gather(n_rows, d, n_idx, k):
    ew = n_idx // NW
    steps = ew // k
    assert ew % k == 0 and k % 8 == 0 and ew * NW == n_idx

    def body(table_hbm, idx_hbm, out_hbm, idx_v, rows_v, sem):
        wid = lax.axis_index("s") * NC + lax.axis_index("c")
        base = wid * ew

        def step(i, c):
            off = base + i * k
            pltpu.sync_copy(idx_hbm.at[pl.ds(off, k)], idx_v)
            pltpu.async_copy(table_hbm.at[idx_v], rows_v, sem).wait()
            pltpu.sync_copy(rows_v, out_hbm.at[pl.ds(off, k)])
            return c

        lax.fori_loop(0, steps, step, 0)

    return pl.kernel(
        body,
        out_type=jax.ShapeDtypeStruct((n_idx, d), _f32),
        mesh=_mesh(),
        compiler_params=pltpu.CompilerParams(use_tc_tiling_on_sc=False),
        scratch_types=[
            pltpu.VMEM((k,), jnp.int32),
            pltpu.VMEM((k, d), _f32),
            pltpu.SemaphoreType.DMA,
        ],
    )


@functools.lru_cache(maxsize=None)
def _sc_msg_scatter(n_rows, d, n_idx, k):
    ew = n_idx // NW
    steps = ew // k
    zr = n_rows // NS
    zc = 25
    zsteps = zr // zc
    assert ew % k == 0 and k % 8 == 0 and zr * NS == n_rows and zr % zc == 0

    assert steps % 2 == 1

    def body(w_hbm, hlin_hbm, src_hbm, dst_hbm, out_hbm,
             sidx0, sidx1, didx0, didx1, x0, x1, wr0, wr1, zbuf, accum,
             isem0, isem1, jsem0, jsem1, gsem0, gsem1, wsem0, wsem1):
        sidx = [sidx0, sidx1]
        didx = [didx0, didx1]
        xrows = [x0, x1]
        wrows = [wr0, wr1]
        isem = [isem0, isem1]
        jsem = [jsem0, jsem1]
        gsem = [gsem0, gsem1]
        wsem = [wsem0, wsem1]
        cid = lax.axis_index("c")
        sid = lax.axis_index("s")
        wid = sid * NC + cid
        base = wid * ew

        def zrow(r, c):
            for j in range(d // 16):
                zbuf[r, pl.ds(j * 16, 16)] = jnp.zeros((16,), _f32)
            return c

        lax.fori_loop(0, zc, zrow, 0)
        for j in range(zsteps):
            pltpu.sync_copy(zbuf, accum.at[pl.ds(sid * zr + j * zc, zc)])
        plsc.subcore_barrier()

        def eidx(i, b):
            off = base + i * k
            pltpu.async_copy(src_hbm.at[pl.ds(off, k)], sidx[b], isem[b])
            pltpu.async_copy(dst_hbm.at[pl.ds(off, k)], didx[b], jsem[b])

        def emain(i, b):
            off = base + i * k
            pltpu.make_async_copy(src_hbm.at[pl.ds(off, k)], sidx[b],
                                  isem[b]).wait()
            pltpu.async_copy(hlin_hbm.at[sidx[b]], xrows[b], gsem[b])
            pltpu.async_copy(w_hbm.at[pl.ds(off, k)], wrows[b], wsem[b])

        def consume(i, b):
            off = base + i * k
            pltpu.make_async_copy(hlin_hbm.at[sidx[b]], xrows[b],
                                  gsem[b]).wait()
            pltpu.make_async_copy(w_hbm.at[pl.ds(off, k)], wrows[b],
                                  wsem[b]).wait()
            pltpu.make_async_copy(dst_hbm.at[pl.ds(off, k)], didx[b],
                                  jsem[b]).wait()

            def mrow(r, c2):
                for j in range(d // 16):
                    sl = pl.ds(j * 16, 16)
                    wrows[b][r, sl] = wrows[b][r, sl] * xrows[b][r, sl]
                return c2

            lax.fori_loop(0, k, mrow, 0)
            pltpu.sync_copy(wrows[b], accum.at[didx[b]], add=True)

        eidx(0, 0)
        eidx(1, 1)
        emain(0, 0)

        def pair(p, c):
            i0 = 2 * p
            emain(i0 + 1, 1)
            consume(i0, 0)
            eidx(i0 + 2, 0)
            i1 = i0 + 1
            emain(i1 + 1, 0)
            consume(i1, 1)

            @pl.when(i1 + 2 < steps)
            def _():
                eidx(i1 + 2, 1)

            return c

        lax.fori_loop(0, (steps - 1) // 2, pair, 0)
        consume(steps - 1, 0)
        plsc.subcore_barrier()

        pltpu.sync_copy(accum.at[pl.ds(sid * zr, zr)],
                        out_hbm.at[cid].at[pl.ds(sid * zr, zr)])

    return pl.kernel(
        body,
        out_type=jax.ShapeDtypeStruct((NC, n_rows, d), _f32),
        mesh=_mesh(),
        compiler_params=pltpu.CompilerParams(use_tc_tiling_on_sc=False),
        scratch_types=[
            pltpu.VMEM((k,), jnp.int32),
            pltpu.VMEM((k,), jnp.int32),
            pltpu.VMEM((k,), jnp.int32),
            pltpu.VMEM((k,), jnp.int32),
            pltpu.VMEM((k, d), _f32),
            pltpu.VMEM((k, d), _f32),
            pltpu.VMEM((k, d), _f32),
            pltpu.VMEM((k, d), _f32),
            pltpu.VMEM((zc, d), _f32),
            pltpu.VMEM_SHARED((n_rows, d), _f32),
            pltpu.SemaphoreType.DMA,
            pltpu.SemaphoreType.DMA,
            pltpu.SemaphoreType.DMA,
            pltpu.SemaphoreType.DMA,
            pltpu.SemaphoreType.DMA,
            pltpu.SemaphoreType.DMA,
            pltpu.SemaphoreType.DMA,
            pltpu.SemaphoreType.DMA,
        ],
    )


@functools.lru_cache(maxsize=None)
def _embed(h):
    r = 1000
    grid = N_NODES // r
    zp = 104

    def body(z_r, emb_r, l1_r, h_r, hlin_r):
        oh = (z_r[...] == lax.broadcasted_iota(jnp.int32, (r, zp), 1)
              ).astype(_f32)
        h0 = jnp.dot(oh, emb_r[...], preferred_element_type=_f32)
        h_r[...] = h0
        hlin_r[...] = jnp.dot(h0, l1_r[...], preferred_element_type=_f32)

    return pl.pallas_call(
        body,
        grid=(grid,),
        in_specs=[
            pl.BlockSpec((r, 1), lambda i: (i, 0)),
            pl.BlockSpec((zp, h), lambda i: (0, 0)),
            pl.BlockSpec((h, h), lambda i: (0, 0)),
        ],
        out_specs=(pl.BlockSpec((r, h), lambda i: (i, 0)),
                   pl.BlockSpec((r, h), lambda i: (i, 0))),
        out_shape=(jax.ShapeDtypeStruct((N_NODES, h), _f32),
                   jax.ShapeDtypeStruct((N_NODES, h), _f32)),
    )


@functools.lru_cache(maxsize=None)
def _edge_wall(hdims):
    te = 2000
    grid = N_EDGES // te
    delta = CUTOFF / (NG - 1)
    coeff = -0.5 / delta ** 2
    nb = len(hdims)

    def body(*refs):
        px_r, py_r = refs[0], refs[1]
        ins = refs[2:2 + 4 * nb]
        outs = refs[2 + 4 * nb:]
        dxyz = px_r[...] - py_r[...]
        d2 = jnp.sum(dxyz * dxyz, axis=1, keepdims=True)
        d = jnp.sqrt(d2 + 1e-12)
        offs = lax.broadcasted_iota(jnp.int32, (te, NGP), 1).astype(_f32) * delta
        dd = d - offs
        rbf = jnp.exp(coeff * (dd * dd))
        c = 0.5 * (jnp.cos(d * (math.pi / CUTOFF)) + 1.0)
        c = jnp.where(d < CUTOFF, c, 0.0)
        for bi in range(nb):
            w1_r, b1_r, w2_r, b2_r = ins[4 * bi:4 * bi + 4]
            t = _ssp(jnp.dot(rbf, w1_r[...], preferred_element_type=_f32)
                     + b1_r[...])
            w = jnp.dot(t, w2_r[...], preferred_element_type=_f32) + b2_r[...]
            outs[bi][...] = w * c

    in_specs = [pl.BlockSpec((te, 16), lambda i: (i, 0)),
                pl.BlockSpec((te, 16), lambda i: (i, 0))]
    for h in hdims:
        in_specs += [
            pl.BlockSpec((NGP, h), lambda i: (0, 0)),
            pl.BlockSpec((1, h), lambda i: (0, 0)),
            pl.BlockSpec((h, h), lambda i: (0, 0)),
            pl.BlockSpec((1, h), lambda i: (0, 0)),
        ]
    return pl.pallas_call(
        body,
        grid=(grid,),
        in_specs=in_specs,
        out_specs=tuple(pl.BlockSpec((te, h), lambda i: (i, 0))
                        for h in hdims),
        out_shape=tuple(jax.ShapeDtypeStruct((N_EDGES, h), _f32)
                        for h in hdims),
    )


@functools.lru_cache(maxsize=None)
def _node_update(h, nxt):
    r = 1000
    grid = N_NODES // r

    def body(a0_r, a1_r, h_r, l2w_r, l2b_r, lw_r, lb_r, *rest):
        agg = a0_r[...] + a1_r[...]
        x = _ssp(jnp.dot(agg, l2w_r[...], preferred_element_type=_f32)
                 + l2b_r[...])
        x = jnp.dot(x, lw_r[...], preferred_element_type=_f32) + lb_r[...]
        hn = h_r[...] + x
        if nxt:
            nw_r, hn_r, hlin_r = rest
            hn_r[...] = hn
            hlin_r[...] = jnp.dot(hn, nw_r[...], preferred_element_type=_f32)
        else:
            (hn_r,) = rest
            hn_r[...] = hn

    in_specs = [
        pl.BlockSpec((r, h), lambda i: (i, 0)),
        pl.BlockSpec((r, h), lambda i: (i, 0)),
        pl.BlockSpec((r, h), lambda i: (i, 0)),
        pl.BlockSpec((h, h), lambda i: (0, 0)),
        pl.BlockSpec((1, h), lambda i: (0, 0)),
        pl.BlockSpec((h, h), lambda i: (0, 0)),
        pl.BlockSpec((1, h), lambda i: (0, 0)),
    ]
    if nxt:
        in_specs.append(pl.BlockSpec((h, h), lambda i: (0, 0)))
        out_specs = (pl.BlockSpec((r, h), lambda i: (i, 0)),
                     pl.BlockSpec((r, h), lambda i: (i, 0)))
        out_shape = (jax.ShapeDtypeStruct((N_NODES, h), _f32),
                     jax.ShapeDtypeStruct((N_NODES, h), _f32))
    else:
        out_specs = pl.BlockSpec((r, h), lambda i: (i, 0))
        out_shape = jax.ShapeDtypeStruct((N_NODES, h), _f32)

    return pl.pallas_call(
        body,
        grid=(grid,),
        in_specs=in_specs,
        out_specs=out_specs,
        out_shape=out_shape,
    )


@functools.lru_cache(maxsize=None)
def _readout(h, with_prev):
    r = 1000
    grid = N_NODES // r
    hh = h // 2

    def body(h_r, w1_r, b1_r, w2_r, b2_r, bt_r, aux_r, out_r):
        i = pl.program_id(0)
        t = _ssp(jnp.dot(h_r[...], w1_r[...], preferred_element_type=_f32)
                 + b1_r[...])
        e = jnp.dot(t, w2_r[...], preferred_element_type=_f32) + b2_r[...]
        oh = (bt_r[...] == lax.broadcasted_iota(jnp.int32, (r, N_MOL), 1)
              ).astype(_f32)
        part = jnp.sum(oh * e, axis=0, keepdims=True)
        if with_prev:
            @pl.when(i == 0)
            def _():
                out_r[...] = aux_r[...]
            out_r[...] += part
        else:
            @pl.when(i == 0)
            def _():
                out_r[...] = jnp.zeros((1, N_MOL), _f32)
            out_r[...] += part * aux_r[0, 0]

    aux_spec = (pl.BlockSpec((1, N_MOL), lambda i: (0, 0)) if with_prev
                else pl.BlockSpec((1, 1), lambda i: (0, 0)))
    return pl.pallas_call(
        body,
        grid=(grid,),
        in_specs=[
            pl.BlockSpec((r, h), lambda i: (i, 0)),
            pl.BlockSpec((h, hh), lambda i: (0, 0)),
            pl.BlockSpec((1, hh), lambda i: (0, 0)),
            pl.BlockSpec((hh, 1), lambda i: (0, 0)),
            pl.BlockSpec((1, 1), lambda i: (0, 0)),
            pl.BlockSpec((r, 1), lambda i: (i, 0)),
            aux_spec,
        ],
        out_specs=pl.BlockSpec((1, N_MOL), lambda i: (0, 0)),
        out_shape=jax.ShapeDtypeStruct((1, N_MOL), _f32),
    )


def _run_model(params, hdim, z2, ws, src, dst, sc_k):
    blocks = params["blocks"]
    t_total = len(blocks)
    emb = jnp.pad(params["emb"], ((0, 4), (0, 0)))
    h, hlin = _embed(hdim)(z2, emb, blocks[0]["lin1_w"])
    for t, blk in enumerate(blocks):
        agg = _sc_msg_scatter(N_NODES, hdim, N_EDGES, sc_k)(
            ws[t], hlin, src, dst)
        a0, a1 = agg[0], agg[1]
        args = (a0, a1, h, blk["lin2_w"], blk["lin2_b"].reshape(1, -1),
                blk["lin_w"], blk["lin_b"].reshape(1, -1))
        if t + 1 < t_total:
            h, hlin = _node_update(hdim, True)(*args,
                                               blocks[t + 1]["lin1_w"])
        else:
            h = _node_update(hdim, False)(*args)
    return h


def kernel(z, pos, edge_index, batch, low_params, dif_params, corr_w):
    src = edge_index[0].astype(jnp.int32)
    dst = edge_index[1].astype(jnp.int32)
    idx2 = jnp.concatenate([src, dst])
    pos16 = jnp.pad(pos.astype(_f32), ((0, 0), (0, 13)))
    pxy = _sc_gather(N_NODES, 16, 2 * N_EDGES, 2000)(pos16, idx2)
    px, py = pxy[:N_EDGES], pxy[N_EDGES:]
    z2 = z.reshape(-1, 1).astype(jnp.int32)
    b2 = batch.reshape(-1, 1).astype(jnp.int32)

    all_blocks = low_params["blocks"] + dif_params["blocks"]
    hdims = tuple(blk["mlp_w2"].shape[0] for blk in all_blocks)
    wall_in = []
    for blk in all_blocks:
        wall_in += [jnp.pad(blk["mlp_w1"], ((0, NGP - NG), (0, 0))),
                    blk["mlp_b1"].reshape(1, -1), blk["mlp_w2"],
                    blk["mlp_b2"].reshape(1, -1)]
    ws = _edge_wall(hdims)(px, py, *wall_in)

    h_low = _run_model(low_params, 128, z2, ws[:3], src, dst, 80)
    h_dif = _run_model(dif_params, 64, z2, ws[3:], src, dst, 80)

    y0 = _readout(128, False)(h_low, low_params["out1_w"],
                              low_params["out1_b"].reshape(1, -1),
                              low_params["out2_w"],
                              low_params["out2_b"].reshape(1, -1),
                              b2, corr_w)
    y = _readout(64, True)(h_dif, dif_params["out1_w"],
                           dif_params["out1_b"].reshape(1, -1),
                           dif_params["out2_w"],
                           dif_params["out2_b"].reshape(1, -1),
                           b2, y0)
    return y.reshape(N_MOL)
